# baseline (speedup 1.0000x reference)
import functools

import jax
import jax.numpy as jnp
from jax import lax
from jax.experimental import pallas as pl
from jax.experimental.pallas import tpu as pltpu
from jax.experimental.pallas import tpu_sc as plsc

BATCH = 16384
EMBED_K = 32
NUM_CORES = 2
NUM_SUBCORES = 16
NUM_WORKERS = NUM_CORES * NUM_SUBCORES
PER_WORKER = BATCH // NUM_WORKERS
CHUNK = 128
NUM_CHUNKS = PER_WORKER // CHUNK
LANES = 16
NUM_GROUPS = PER_WORKER // LANES


def _sc_body(w_hbm, h_hbm, u_hbm, v_hbm, out_hbm,
             u_idx, v_idx, u_rows, v_rows, out_v, sem):
    wid = lax.axis_index("c") * NUM_SUBCORES + lax.axis_index("s")
    base = wid * PER_WORKER

    pltpu.sync_copy(u_hbm.at[wid], u_idx)
    pltpu.sync_copy(v_hbm.at[wid], v_idx)

    copies = []
    for c in range(NUM_CHUNKS):
        rows_sl = pl.ds(c * CHUNK, CHUNK)
        copies.append(pltpu.async_copy(w_hbm.at[u_idx.at[c]],
                                       u_rows.at[rows_sl], sem))
        copies.append(pltpu.async_copy(h_hbm.at[v_idx.at[c]],
                                       v_rows.at[rows_sl], sem))
    for cp in copies:
        cp.wait()

    lane = lax.iota(jnp.int32, LANES)

    def group(j, carry):
        rows = lane + j * LANES
        acc = jnp.zeros((LANES,), jnp.float32)
        for k in range(EMBED_K):
            col = jnp.full((LANES,), k, jnp.int32)
            uk = plsc.load_gather(u_rows, [rows, col])
            vk = plsc.load_gather(v_rows, [rows, col])
            acc = acc + uk * vk
        res = 1.0 / (1.0 + jnp.exp(-acc))
        out_v[pl.ds(pl.multiple_of(j * LANES, LANES), LANES)] = res
        return carry

    lax.fori_loop(0, NUM_GROUPS, group, 0, unroll=2)

    pltpu.sync_copy(out_v, out_hbm.at[pl.ds(base, PER_WORKER)])


@jax.jit
def kernel(x, W, H):
    u = x[:, 0].astype(jnp.int32).reshape(NUM_WORKERS, NUM_CHUNKS, CHUNK)
    v = x[:, 1].astype(jnp.int32).reshape(NUM_WORKERS, NUM_CHUNKS, CHUNK)
    mesh = plsc.VectorSubcoreMesh(core_axis_name="c", subcore_axis_name="s")
    run = pl.kernel(
        _sc_body,
        out_type=jax.ShapeDtypeStruct((BATCH,), jnp.float32),
        mesh=mesh,
        scratch_types=[
            pltpu.VMEM((NUM_CHUNKS, CHUNK), jnp.int32),
            pltpu.VMEM((NUM_CHUNKS, CHUNK), jnp.int32),
            pltpu.VMEM((PER_WORKER, EMBED_K), jnp.float32),
            pltpu.VMEM((PER_WORKER, EMBED_K), jnp.float32),
            pltpu.VMEM((PER_WORKER,), jnp.float32),
            pltpu.SemaphoreType.DMA,
        ],
        compiler_params=pltpu.CompilerParams(needs_layout_passes=False,
                                             use_tc_tiling_on_sc=False),
    )
    return run(W, H, u, v)
